# baseline (speedup 1.0000x reference)
import jax
import jax.numpy as jnp
from jax import lax
from jax.experimental import pallas as pl
from jax.experimental.pallas import tpu as pltpu
from jax.experimental.pallas import tpu_sc as plsc

CLASS_NUM = 20
CONF_THRESH = 0.01
IOU_THRESHOLD = 0.45
TOP_K = 200
IMG_W = 512.0
BATCH = 8
N_BOXES = 20000
FEAT = 3 * CLASS_NUM + 5

N_PAD = 20480
NEG = -1e30

L = 16
NG1 = N_PAD // (L * L)
NG2 = NG1 // L
KEPT_PAD = 224
OUT_W = 6
OUT_FLAT = TOP_K * OUT_W

BPC = BATCH // 2
WPB = 4
DEC_PER_W = N_PAD // WPB
DEC_CH = 160
DEC_NCH = DEC_PER_W // DEC_CH
DEC_NCH_LAST = (N_BOXES - (WPB - 1) * DEC_PER_W) // DEC_CH
SFEAT = FEAT - CLASS_NUM
P2OFF = CLASS_NUM + 1
BOXOFF = 2 * CLASS_NUM + 1
ST0 = DEC_CH * SFEAT


def _sc_body(y_hbm, out_hbm, dec_hbm,
             conf_v, cls_v, x1_v, y1_v, x2_v, y2_v, l1_v, l2_v,
             kx1_v, ky1_v, kx2_v, ky2_v, kar_v, out_v, dsemA, dsemB, ssem):
    c = lax.axis_index("c")
    s = lax.axis_index("s")
    iota = jnp.arange(L, dtype=jnp.int32)

    lb = s // WPB
    q = s - lb * WPB
    b = BPC * c + lb
    n0 = q * DEC_PER_W
    nch = jnp.where(q == WPB - 1, DEC_NCH_LAST, DEC_NCH)

    CHW = DEC_CH * SFEAT

    def fire(ci, yref, base, sem):
        start = n0 + ci * DEC_CH
        pltpu.make_async_copy(y_hbm.at[b, pl.ds(start * SFEAT, CHW)],
                              yref.at[pl.ds(base, CHW)], sem).start()

    def compute(ci, yref, base):
        start = n0 + ci * DEC_CH

        def feat(rows, f):
            return plsc.load_gather(yref, [base + rows * SFEAT + f])

        for g in range(DEC_CH // L):
            rows = g * L + iota
            best = feat(rows, 0) * feat(rows, P2OFF)
            bid = jnp.zeros((L,), jnp.float32)
            for cc in range(1, CLASS_NUM):
                pc = (feat(rows, cc)
                      * feat(rows, P2OFF + cc))
                upd = pc > best
                best = jnp.where(upd, pc, best)
                bid = jnp.where(upd, jnp.float32(cc), bid)

            def clipv(v):
                return jnp.maximum(jnp.minimum(v, IMG_W - 1.0), 0.0)

            o = ST0 + g * L
            x1_v[pl.ds(o, L)] = best
            x1_v[pl.ds(o + DEC_CH, L)] = bid + 1.0
            x1_v[pl.ds(o + 2 * DEC_CH, L)] = clipv(feat(rows, BOXOFF))
            x1_v[pl.ds(o + 3 * DEC_CH, L)] = clipv(feat(rows, BOXOFF + 1))
            x1_v[pl.ds(o + 4 * DEC_CH, L)] = clipv(feat(rows, BOXOFF + 2))
            x1_v[pl.ds(o + 5 * DEC_CH, L)] = clipv(feat(rows, BOXOFF + 3))

        for a in range(6):
            pltpu.sync_copy(x1_v.at[pl.ds(ST0 + a * DEC_CH, DEC_CH)],
                            dec_hbm.at[a, b, pl.ds(start, DEC_CH)])

    fire(0, cls_v, 0, dsemA)

    def chunk(ci, _):
        def even():
            start = n0 + ci * DEC_CH
            pltpu.make_async_copy(y_hbm.at[b, pl.ds(start * SFEAT, CHW)],
                                  cls_v.at[pl.ds(0, CHW)], dsemA).wait()

            @pl.when(ci + 1 < nch)
            def _f():
                fire(ci + 1, y1_v, 0, dsemB)
            compute(ci, cls_v, 0)

        def odd():
            start = n0 + ci * DEC_CH
            pltpu.make_async_copy(y_hbm.at[b, pl.ds(start * SFEAT, CHW)],
                                  y1_v.at[pl.ds(0, CHW)], dsemB).wait()

            @pl.when(ci + 1 < nch)
            def _f():
                fire(ci + 1, cls_v, 0, dsemA)
            compute(ci, y1_v, 0)

        lax.cond(ci % 2 == 0, even, odd)
        return 0

    lax.fori_loop(0, nch, chunk, 0)
    plsc.subcore_barrier()

    @pl.when(s < BPC)
    def _run():
        gb = BPC * c + s
        arrs = (conf_v, cls_v, x1_v, y1_v, x2_v, y2_v)
        for a, dstv in enumerate(arrs):
            pltpu.make_async_copy(dec_hbm.at[a, gb], dstv, ssem).start()
        for a, dstv in enumerate(arrs):
            pltpu.make_async_copy(dec_hbm.at[a, gb], dstv, ssem).wait()

        zf = jnp.zeros((L,), jnp.float32)
        lane0 = iota == 0
        negv = jnp.full((L,), NEG, jnp.float32)

        for i in range((N_PAD - N_BOXES) // L):
            conf_v[pl.ds(N_BOXES + i * L, L)] = negv

        def _zero(i, _):
            out_v[pl.ds(i * L, L)] = zf
            return 0
        lax.fori_loop(0, OUT_FLAT // L, _zero, 0)
        for w in range(KEPT_PAD // L):
            kx1_v[pl.ds(w * L, L)] = zf
            ky1_v[pl.ds(w * L, L)] = zf
            kx2_v[pl.ds(w * L, L)] = zf
            ky2_v[pl.ds(w * L, L)] = zf
            kar_v[pl.ds(w * L, L)] = zf

        def _build_l1(g, _):
            acc = negv
            base = g * (L * L)
            for k in range(L):
                v = conf_v[pl.ds(base + k * L, L)]
                v = jnp.where(v > CONF_THRESH, v, NEG)
                conf_v[pl.ds(base + k * L, L)] = v
                acc = jnp.maximum(acc, v)
            l1_v[pl.ds(g * L, L)] = acc
            return 0
        lax.fori_loop(0, NG1, _build_l1, 0)

        for h in range(NG2):
            acc = negv
            for k in range(L):
                acc = jnp.maximum(acc, l1_v[pl.ds((h * L + k) * L, L)])
            l2_v[pl.ds(h * L, L)] = acc

        def top_max():
            acc = l2_v[pl.ds(0, L)]
            for h in range(1, NG2):
                acc = jnp.maximum(acc, l2_v[pl.ds(h * L, L)])
            return jnp.max(acc)

        def cond(st):
            kcnt, m = st
            return (kcnt < TOP_K) & (m > CONF_THRESH)

        def body(st):
            kcnt, m = st
            msp = jnp.full((L,), m)

            hbest = jnp.full((L,), NG2, jnp.int32)
            for h in range(NG2):
                vec = l2_v[pl.ds(h * L, L)]
                hbest = jnp.minimum(hbest, jnp.where(vec == msp, h, NG2))
            hstar = jnp.min(hbest)

            kbest = jnp.full((L,), L, jnp.int32)
            for k in range(L):
                vec = l1_v[pl.ds((hstar * L + k) * L, L)]
                kbest = jnp.minimum(kbest, jnp.where(vec == msp, k, L))
            gstar = hstar * L + jnp.min(kbest)

            kbest2 = jnp.full((L,), L, jnp.int32)
            for k in range(L):
                vec = conf_v[pl.ds((gstar * L + k) * L, L)]
                kbest2 = jnp.minimum(kbest2, jnp.where(vec == msp, k, L))
            istar = gstar * L + jnp.min(kbest2)

            civ = conf_v[pl.ds(istar * L, L)]
            jstar = jnp.min(jnp.where(civ == msp, iota, L))
            bstar = istar * L + jstar
            bsp = jnp.full((L,), bstar, jnp.int32)

            sx1 = plsc.load_gather(x1_v, [bsp])
            sy1 = plsc.load_gather(y1_v, [bsp])
            sx2 = plsc.load_gather(x2_v, [bsp])
            sy2 = plsc.load_gather(y2_v, [bsp])
            scl = plsc.load_gather(cls_v, [bsp])
            sar = (jnp.maximum(sx2 - sx1, 0.0) * jnp.maximum(sy2 - sy1, 0.0))

            supm = jnp.zeros((L,), jnp.bool_)
            for w in range(KEPT_PAD // L):
                kx1 = kx1_v[pl.ds(w * L, L)]
                ky1 = ky1_v[pl.ds(w * L, L)]
                kx2 = kx2_v[pl.ds(w * L, L)]
                ky2 = ky2_v[pl.ds(w * L, L)]
                kar = kar_v[pl.ds(w * L, L)]
                xx1 = jnp.maximum(kx1, sx1)
                yy1 = jnp.maximum(ky1, sy1)
                xx2 = jnp.minimum(kx2, sx2)
                yy2 = jnp.minimum(ky2, sy2)
                inter = (jnp.maximum(xx2 - xx1, 0.0)
                         * jnp.maximum(yy2 - yy1, 0.0))
                union = kar + sar - inter
                sv = (union > 0.0) & (inter > IOU_THRESHOLD
                                      * jnp.maximum(union, 1e-12))
                supm = supm | sv
            keepb = jnp.max(supm.astype(jnp.int32)) == 0

            ksp = jnp.full((L,), kcnt, jnp.int32)
            appm = lane0 & keepb
            plsc.store_scatter(kx1_v, [ksp], sx1, mask=appm)
            plsc.store_scatter(ky1_v, [ksp], sy1, mask=appm)
            plsc.store_scatter(kx2_v, [ksp], sx2, mask=appm)
            plsc.store_scatter(ky2_v, [ksp], sy2, mask=appm)
            plsc.store_scatter(kar_v, [ksp], sar, mask=appm)

            row = jnp.where(iota == 0, scl,
                  jnp.where(iota == 1, msp,
                  jnp.where(iota == 2, sx1,
                  jnp.where(iota == 3, sy1,
                  jnp.where(iota == 4, sx2, sy2)))))
            owm = (iota < 6) & keepb
            plsc.store_scatter(out_v, [ksp * OUT_W + iota], row, mask=owm)

            plsc.store_scatter(conf_v, [bsp], negv, mask=lane0)
            l1slot = gstar * L + jstar
            gv = plsc.load_gather(conf_v, [(gstar * L + iota) * L + jstar])
            plsc.store_scatter(l1_v, [jnp.full((L,), l1slot, jnp.int32)],
                               jnp.full((L,), jnp.max(gv)), mask=lane0)
            l2slot = hstar * L + jstar
            hv = plsc.load_gather(l1_v, [(hstar * L + iota) * L + jstar])
            plsc.store_scatter(l2_v, [jnp.full((L,), l2slot, jnp.int32)],
                               jnp.full((L,), jnp.max(hv)), mask=lane0)

            return (kcnt + jnp.where(keepb, 1, 0), top_max())

        lax.while_loop(cond, body, (jnp.int32(0), top_max()))
        pltpu.sync_copy(out_v, out_hbm.at[BPC * c + s])


@jax.jit
def kernel(y_pred):
    run = pl.kernel(
        _sc_body,
        out_type=[jax.ShapeDtypeStruct((BATCH, OUT_FLAT), jnp.float32),
                  jax.ShapeDtypeStruct((6, BATCH, N_PAD), jnp.float32)],
        mesh=plsc.VectorSubcoreMesh(core_axis_name="c", subcore_axis_name="s",
                                    num_cores=2, num_subcores=16),
        compiler_params=pltpu.CompilerParams(needs_layout_passes=False,
                                             use_tc_tiling_on_sc=False),
        scratch_types=[
            pltpu.VMEM((N_PAD,), jnp.float32),
            pltpu.VMEM((N_PAD,), jnp.float32),
            pltpu.VMEM((N_PAD,), jnp.float32),
            pltpu.VMEM((N_PAD,), jnp.float32),
            pltpu.VMEM((N_PAD,), jnp.float32),
            pltpu.VMEM((N_PAD,), jnp.float32),
            pltpu.VMEM((NG1 * L,), jnp.float32),
            pltpu.VMEM((NG2 * L,), jnp.float32),
            pltpu.VMEM((KEPT_PAD,), jnp.float32),
            pltpu.VMEM((KEPT_PAD,), jnp.float32),
            pltpu.VMEM((KEPT_PAD,), jnp.float32),
            pltpu.VMEM((KEPT_PAD,), jnp.float32),
            pltpu.VMEM((KEPT_PAD,), jnp.float32),
            pltpu.VMEM((OUT_FLAT,), jnp.float32),
            pltpu.SemaphoreType.DMA,
            pltpu.SemaphoreType.DMA,
            pltpu.SemaphoreType.DMA,
        ],
    )
    ysl = y_pred[:, :, CLASS_NUM:]
    out, _ = run(ysl.reshape(BATCH, N_BOXES * SFEAT))
    return out.reshape(BATCH, TOP_K, OUT_W)
